# baseline (speedup 1.0000x reference)
import dataclasses
import functools
import jax
import jax.numpy as jnp
from jax import lax
from jax.experimental import pallas as pl
from jax.experimental.pallas import tpu as pltpu
from jax.experimental.pallas import tpu_sc as plsc

HID = 256
HEADS = 4
DH = 64
WMSG = 384
WAGG = 272
NWORK = 32
CGATH = 128


def _mm_body(x_ref, w_ref, b_ref, o_ref, *, act):
    y = jnp.dot(x_ref[...], w_ref[...], preferred_element_type=jnp.float32)
    y = y + b_ref[...]
    if act == 'relu':
        y = jnp.maximum(y, 0.0)
    o_ref[...] = y


def _matmul(x, w, b, act=None, bm=1000):
    M, K = x.shape
    N = w.shape[1]
    return pl.pallas_call(
        functools.partial(_mm_body, act=act),
        grid=(M // bm,),
        in_specs=[pl.BlockSpec((bm, K), lambda i: (i, 0)),
                  pl.BlockSpec((K, N), lambda i: (0, 0)),
                  pl.BlockSpec((1, N), lambda i: (0, 0))],
        out_specs=pl.BlockSpec((bm, N), lambda i: (i, 0)),
        out_shape=jax.ShapeDtypeStruct((M, N), jnp.float32),
    )(x, w, b.reshape(1, N))


def _mm5_body(x_ref, w_ref, b_ref, o0, o1, o2):
    y = jnp.dot(x_ref[...], w_ref[...], preferred_element_type=jnp.float32)
    y = y + b_ref[...]
    o0[...] = y[:, 0:2 * HID].astype(jnp.bfloat16)
    o1[...] = y[:, 2 * HID:4 * HID].astype(jnp.bfloat16)
    o2[...] = y[:, 4 * HID:5 * HID]


def _matmul5(x, w, b, bm=1000):
    M, K = x.shape
    kv = jax.ShapeDtypeStruct((M, 2 * HID), jnp.bfloat16)
    return pl.pallas_call(
        _mm5_body,
        grid=(M // bm,),
        in_specs=[pl.BlockSpec((bm, K), lambda i: (i, 0)),
                  pl.BlockSpec((K, 5 * HID), lambda i: (0, 0)),
                  pl.BlockSpec((1, 5 * HID), lambda i: (0, 0))],
        out_specs=[pl.BlockSpec((bm, 2 * HID), lambda i: (i, 0)),
                   pl.BlockSpec((bm, 2 * HID), lambda i: (i, 0)),
                   pl.BlockSpec((bm, HID), lambda i: (i, 0))],
        out_shape=[kv, kv, jax.ShapeDtypeStruct((M, HID), jnp.float32)],
    )(x, w, b.reshape(1, 5 * HID))


def _msg_body(gkv_ref, gq_ref, d_ref, s_ref, r_ref, o_ref):
    gk = gkv_ref[:, :HID].astype(jnp.float32)
    gq = gq_ref[...]
    gv = gkv_ref[:, HID:].astype(jnp.float32)
    s = jnp.dot(gk * gq, s_ref[...], preferred_element_type=jnp.float32)
    e16 = jnp.exp(s)
    rep = jnp.dot(e16, r_ref[...], preferred_element_type=jnp.float32)
    df = lax.bitcast_convert_type(d_ref[...], jnp.float32)
    zpad = jnp.zeros((gk.shape[0], WMSG - HID - 17), jnp.float32)
    o_ref[...] = jnp.concatenate([gv * rep, e16, df, zpad], axis=1)


def _msg_kernel(gkv, gq, dst, smat, rmat, bm=512):
    E = gkv.shape[0]
    return pl.pallas_call(
        _msg_body,
        grid=(E // bm,),
        in_specs=[pl.BlockSpec((bm, 2 * HID), lambda i: (i, 0)),
                  pl.BlockSpec((bm, HID), lambda i: (i, 0)),
                  pl.BlockSpec((bm, 1), lambda i: (i, 0)),
                  pl.BlockSpec((HID, 16), lambda i: (0, 0)),
                  pl.BlockSpec((16, HID), lambda i: (0, 0))],
        out_specs=pl.BlockSpec((bm, WMSG), lambda i: (i, 0)),
        out_shape=jax.ShapeDtypeStruct((E, WMSG), jnp.float32),
    )(gkv, gq, dst.reshape(E, 1), smat, rmat)


def _outproj_body(agg_ref, h_ref, w_ref, b_ref, bv_ref, r4_ref, o_ref):
    a = agg_ref[...]
    den = a[:, HID:HID + 4]
    wgt = 1.0 / (den + 1e-16)
    wrep = jnp.dot(wgt, r4_ref[...], preferred_element_type=jnp.float32)
    x = a[:, :HID] * wrep
    g = jax.nn.gelu(x)
    o = jnp.dot(g, w_ref[...], preferred_element_type=jnp.float32) + b_ref[...]
    o_ref[...] = jnp.maximum(o + h_ref[...] * bv_ref[...], 0.0)


def _outproj(agg2, h_prev, w, b, bvec, r4, bm=1000):
    N = h_prev.shape[0]
    return pl.pallas_call(
        _outproj_body,
        grid=(N // bm,),
        in_specs=[pl.BlockSpec((bm, WAGG), lambda i: (i, 0)),
                  pl.BlockSpec((bm, HID), lambda i: (i, 0)),
                  pl.BlockSpec((HID, HID), lambda i: (0, 0)),
                  pl.BlockSpec((1, HID), lambda i: (0, 0)),
                  pl.BlockSpec((1, HID), lambda i: (0, 0)),
                  pl.BlockSpec((4, HID), lambda i: (0, 0))],
        out_specs=pl.BlockSpec((bm, HID), lambda i: (i, 0)),
        out_shape=jax.ShapeDtypeStruct((N, HID), jnp.float32),
    )(agg2, h_prev, w, b.reshape(1, HID), bvec.reshape(1, HID), r4)


def _head_body(ga_ref, gb_ref, ea_ref, wc_ref, b1_ref, w2_ref, b2_ref,
               w3_ref, b3_ref, o_ref, *, has_ea):
    z = ga_ref[...] + gb_ref[...] + b1_ref[...]
    if has_ea:
        z = z + jnp.dot(ea_ref[...], wc_ref[...],
                        preferred_element_type=jnp.float32)
    h = jnp.maximum(z, 0.0)
    h2 = jnp.dot(h, w2_ref[...], preferred_element_type=jnp.float32)
    h2 = jnp.maximum(h2 + b2_ref[...], 0.0)
    y = jnp.dot(h2, w3_ref[...], preferred_element_type=jnp.float32)
    o_ref[...] = y + b3_ref[...]


def _head(ga, gb, ea, wc, b1, w2, b2, w3, b3, bm=512):
    E = ga.shape[0]
    has_ea = ea is not None
    if not has_ea:
        ea = jnp.zeros((E, 8), jnp.float32)
        wc = jnp.zeros((8, HID), jnp.float32)
    return pl.pallas_call(
        functools.partial(_head_body, has_ea=has_ea),
        grid=(E // bm,),
        in_specs=[pl.BlockSpec((bm, HID), lambda i: (i, 0)),
                  pl.BlockSpec((bm, HID), lambda i: (i, 0)),
                  pl.BlockSpec((bm, 8), lambda i: (i, 0)),
                  pl.BlockSpec((8, HID), lambda i: (0, 0)),
                  pl.BlockSpec((1, HID), lambda i: (0, 0)),
                  pl.BlockSpec((HID, 128), lambda i: (0, 0)),
                  pl.BlockSpec((1, 128), lambda i: (0, 0)),
                  pl.BlockSpec((128, 1), lambda i: (0, 0)),
                  pl.BlockSpec((1, 1), lambda i: (0, 0))],
        out_specs=pl.BlockSpec((bm, 1), lambda i: (i, 0)),
        out_shape=jax.ShapeDtypeStruct((E, 1), jnp.float32),
    )(ga, gb, ea, wc, b1.reshape(1, HID), w2, b2.reshape(1, 128),
      w3, b3.reshape(1, 1))


def _sc_gather(table, idx):
    E = idx.shape[0]
    W = table.shape[1]
    dt = table.dtype
    cg = 128
    rows = E // NWORK
    nch = rows // cg
    assert rows % cg == 0 and nch % 2 == 0 and W % 128 == 0
    mesh = plsc.VectorSubcoreMesh(core_axis_name="c", subcore_axis_name="s")

    @functools.partial(
        pl.kernel, mesh=mesh,
        out_type=jax.ShapeDtypeStruct((E, W), dt),
        scratch_types=[
            pltpu.VMEM((rows,), jnp.int32),
            pltpu.VMEM((cg, W), dt),
            pltpu.VMEM((cg, W), dt),
            pltpu.SemaphoreType.DMA,
            pltpu.SemaphoreType.DMA,
            pltpu.SemaphoreType.DMA,
            pltpu.SemaphoreType.DMA,
        ],
    )
    def k(table_hbm, idx_hbm, out_hbm, idx_v, buf_a, buf_b, ga, gb, wa, wb):
        wid = lax.axis_index("s") * 2 + lax.axis_index("c")
        base = wid * rows
        pltpu.sync_copy(idx_hbm.at[pl.ds(base, rows)], idx_v)

        def start_g(j, buf, sem):
            pltpu.async_copy(
                table_hbm.at[idx_v.at[pl.ds(j * cg, cg)]], buf, sem)

        def wait_g(buf, sem):
            pltpu.make_async_copy(
                table_hbm.at[idx_v.at[pl.ds(0, cg)]], buf, sem).wait()

        def start_w(j, buf, sem):
            pltpu.async_copy(buf, out_hbm.at[pl.ds(base + j * cg, cg)], sem)

        def wait_w(buf, sem):
            pltpu.make_async_copy(
                buf, out_hbm.at[pl.ds(base, cg)], sem).wait()

        start_g(0, buf_a, ga)
        start_g(1, buf_b, gb)

        @pl.loop(0, nch // 2)
        def _(jj):
            j = jj * 2
            wait_g(buf_a, ga)
            start_w(j, buf_a, wa)
            wait_g(buf_b, gb)
            start_w(j + 1, buf_b, wb)
            wait_w(buf_a, wa)
            start_g(lax.rem(j + 2, nch), buf_a, ga)
            wait_w(buf_b, wb)
            start_g(lax.rem(j + 3, nch), buf_b, gb)

        wait_g(buf_a, ga)
        wait_g(buf_b, gb)

    return k(table, idx)


def _sc_segsum(msg, dst, nseg_pad, wrows, cap=8320):
    E = msg.shape[0]
    SC_CH = 2048
    nsc = E // SC_CH
    npass = nseg_pad // (NWORK * wrows)
    assert E % SC_CH == 0 and nseg_pad % (NWORK * wrows) == 0
    assert cap % 16 == 0
    mesh = plsc.VectorSubcoreMesh(core_axis_name="c", subcore_axis_name="s")
    cp = pltpu.CompilerParams()
    if "needs_layout_passes" in pltpu.CompilerParams.__dataclass_fields__:
        cp = dataclasses.replace(cp, needs_layout_passes=False)

    @functools.partial(
        pl.kernel, mesh=mesh, compiler_params=cp,
        out_type=jax.ShapeDtypeStruct((nseg_pad, WAGG), jnp.float32),
        scratch_types=[
            pltpu.VMEM((SC_CH,), jnp.int32),
            pltpu.VMEM((cap,), jnp.int32),
            pltpu.VMEM((CGATH, WMSG), jnp.float32),
            pltpu.VMEM((wrows, WAGG), jnp.float32),
            pltpu.SemaphoreType.DMA,
        ],
    )
    def k(msg_hbm, dst_hbm, out_hbm, sbuf, ids, gbuf, acc, gsem):
        w = lax.axis_index("s") * 2 + lax.axis_index("c")
        lane = lax.iota(jnp.int32, 16)
        zf16 = jnp.zeros((16,), jnp.float32)
        zi16 = jnp.zeros((16,), jnp.int32)

        @pl.loop(0, npass)
        def _(p):
            d0 = (p * NWORK + w) * wrows

            @pl.loop(0, wrows)
            def _(r):
                for t in range(WAGG // 16):
                    acc[r, pl.ds(t * 16, 16)] = zf16

            @pl.loop(0, cap // 16)
            def _(r):
                ids[pl.ds(r * 16, 16)] = zi16

            def scan_chunk(c, off):
                pltpu.sync_copy(dst_hbm.at[pl.ds(c * SC_CH, SC_CH)], sbuf)

                def scan_vec(t, off):
                    v = sbuf[pl.ds(t * 16, 16)]
                    m = (v >= d0) & (v < d0 + wrows)
                    eid = (c * SC_CH + t * 16) + lane
                    plsc.store_compressed(ids.at[pl.ds(off, 16)], eid, mask=m)
                    pc = plsc.all_reduce_population_count(m)
                    return off + jnp.max(pc)

                return lax.fori_loop(0, SC_CH // 16, scan_vec, off)

            off = lax.fori_loop(0, nsc, scan_chunk, jnp.int32(0))

            def gchunk(gc, _):
                pltpu.async_copy(
                    msg_hbm.at[ids.at[pl.ds(gc * CGATH, CGATH)]], gbuf,
                    gsem).wait()
                cnt = jnp.minimum(off - gc * CGATH, CGATH)

                def acc_edge(i, _):
                    q = gc * CGATH + i
                    dv = lax.bitcast_convert_type(
                        gbuf[i, pl.ds(HID + 16, 16)], jnp.int32)
                    d = jnp.max(jnp.where(lane == 0, dv, 0)) - d0
                    for t in range(WAGG // 16):
                        plsc.addupdate(acc.at[d, pl.ds(t * 16, 16)],
                                       gbuf[i, pl.ds(t * 16, 16)])
                    return 0

                lax.fori_loop(0, cnt, acc_edge, 0)
                return 0

            lax.fori_loop(0, (off + CGATH - 1) // CGATH, gchunk, 0)
            pltpu.sync_copy(acc, out_hbm.at[pl.ds(d0, wrows)])

    return k(msg, dst)


def _block_diag4(a):
    out = jnp.zeros((HID, HID), jnp.float32)
    for h in range(HEADS):
        out = out.at[h * DH:(h + 1) * DH, h * DH:(h + 1) * DH].set(a[h])
    return out


def _pack32(x):
    m, n = x.shape
    return lax.bitcast_convert_type(x.reshape(m, n // 2, 2), jnp.float32)


def _unpack32(x):
    m, n = x.shape
    return lax.bitcast_convert_type(x, jnp.bfloat16).reshape(m, 2 * n)


def _pad_idx(idx, epad, fill):
    return jnp.pad(idx, (0, epad - idx.shape[0]), constant_values=fill)


def kernel(x_sentence, x_criterion, x_post, edge_attr_supports, supports_src,
           supports_dst, next_src, next_dst, matches_src, matches_dst, params):
    p = params
    NS = x_sentence.shape[0]
    NC = x_criterion.shape[0]
    ES = supports_src.shape[0]
    EN = next_src.shape[0]
    EM = matches_src.shape[0]
    ES_P = 204800
    EN_P = 57344

    f32 = jnp.float32
    smat = (jnp.arange(HID)[:, None] // DH ==
            jnp.arange(16)[None, :]).astype(f32)
    rmat = (jnp.arange(16)[:, None] ==
            jnp.arange(HID)[None, :] // DH).astype(f32)
    r4 = rmat[:4]

    wq = {}
    wkv = {}
    wout = {}
    for l in range(2):
        for (edge, dt) in (('supports', 'criterion'), ('next', 'sentence')):
            scale = jnp.repeat(p[f'pr_{edge}_{l}'] / 8.0, DH)
            wq[(dt, l)] = (p[f'Wq_{dt}_{l}'] * scale[None, :],
                           p[f'bq_{dt}_{l}'] * scale)
        for edge in ('supports', 'next'):
            akb = _block_diag4(p[f'Ak_{edge}_{l}'])
            avb = _block_diag4(p[f'Av_{edge}_{l}'])
            wkv[(edge, 'k', l)] = (p[f'Wk_sentence_{l}'] @ akb,
                                   p[f'bk_sentence_{l}'] @ akb)
            wkv[(edge, 'v', l)] = (p[f'Wv_sentence_{l}'] @ avb,
                                   p[f'bv_sentence_{l}'] @ avb)
        for t in ('sentence', 'criterion'):
            alpha = jax.nn.sigmoid(p[f'skip_{t}_{l}'])
            wout[(t, l)] = (alpha * p[f'Wout_{t}_{l}'],
                            alpha * p[f'bout_{t}_{l}'],
                            (1.0 - alpha) * jnp.ones((HID,), f32))

    wcat = {}
    for l in range(2):
        ws = [wkv[('supports', 'k', l)], wkv[('supports', 'v', l)],
              wkv[('next', 'k', l)], wkv[('next', 'v', l)],
              wq[('sentence', l)]]
        wcat[l] = (jnp.concatenate([w for w, _ in ws], axis=1),
                   jnp.concatenate([b for _, b in ws], axis=0))

    s_src_g = _pad_idx(supports_src, ES_P, 0)
    s_dst_g = _pad_idx(supports_dst, ES_P, 0)
    s_dst_s = _pad_idx(supports_dst, ES_P, 2048)
    n_src_g = _pad_idx(next_src, EN_P, 0)
    n_dst_g = _pad_idx(next_dst, EN_P, 0)
    n_dst_s = _pad_idx(next_dst, EN_P, 51200)
    m_src_g = _pad_idx(matches_src, EN_P, 0)
    m_dst_g = _pad_idx(matches_dst, EN_P, 0)
    ea_pad = jnp.pad(edge_attr_supports, ((0, ES_P - ES), (0, 3)))

    h_s = _matmul(x_sentence, p['Wlin_sentence'], p['blin_sentence'], 'relu')
    h_c = _matmul(x_criterion, p['Wlin_criterion'], p['blin_criterion'], 'relu')
    h_p = _matmul(x_post, p['Wlin_post'], p['blin_post'], 'relu')

    for l in range(2):
        kv_sup, kv_nxt, q_s = _matmul5(h_s, *wcat[l])
        q_c = _matmul(h_c, *wq[('criterion', l)])

        gkv = _unpack32(_sc_gather(_pack32(kv_sup), s_src_g))
        gq = _sc_gather(q_c, s_dst_g)
        msg_s = _msg_kernel(gkv, gq, s_dst_s, smat, rmat)
        agg_c = _sc_segsum(msg_s, s_dst_s, 2048, 64)

        gkv2 = _unpack32(_sc_gather(_pack32(kv_nxt), n_src_g))
        gq2 = _sc_gather(q_s, n_dst_g)
        msg_n = _msg_kernel(gkv2, gq2, n_dst_s, smat, rmat)
        agg_s = _sc_segsum(msg_n, n_dst_s, 51200, 160)

        h_c = _outproj(agg_c, h_c, *wout[('criterion', l)], r4)
        h_s = _outproj(agg_s, h_s, *wout[('sentence', l)], r4)

    we1a = p['We1'][:HID]
    we1b = p['We1'][HID:2 * HID]
    we1c = jnp.pad(p['We1'][2 * HID:], ((0, 3), (0, 0)))
    zb = jnp.zeros((HID,), f32)
    ea_a = _matmul(h_s, we1a, zb)
    ea_b = _matmul(h_c, we1b, zb)
    ga = _sc_gather(ea_a, s_src_g)
    gb = _sc_gather(ea_b, s_dst_g)
    edge_out = _head(ga, gb, ea_pad, we1c, p['be1'], p['We2'], p['be2'],
                     p['We3'], p['be3'])
    edge_logits = edge_out[:ES, 0]

    na_a = _matmul(h_p, p['Wn1'][:HID], zb)
    na_b = _matmul(h_c, p['Wn1'][HID:], zb)
    gp = _sc_gather(na_a, m_src_g)
    gc = _sc_gather(na_b, m_dst_g)
    node_out = _head(gp, gc, None, None, p['bn1'], p['Wn2'], p['bn2'],
                     p['Wn3'], p['bn3'])
    node_logits = node_out[:EM, 0]

    return (edge_logits, node_logits)


# speedup vs baseline: 1.6432x; 1.6432x over previous
import dataclasses
import functools
import jax
import jax.numpy as jnp
from jax import lax
from jax.experimental import pallas as pl
from jax.experimental.pallas import tpu as pltpu
from jax.experimental.pallas import tpu_sc as plsc

HID = 256
HEADS = 4
DH = 64
WMSG = 384
WAGG = 272
NWORK = 32
CGATH = 128


def _mm_body(x_ref, w_ref, b_ref, o_ref, *, act):
    y = jnp.dot(x_ref[...], w_ref[...], preferred_element_type=jnp.float32)
    y = y + b_ref[...]
    if act == 'relu':
        y = jnp.maximum(y, 0.0)
    o_ref[...] = y


def _matmul(x, w, b, act=None, bm=1000):
    M, K = x.shape
    N = w.shape[1]
    return pl.pallas_call(
        functools.partial(_mm_body, act=act),
        grid=(M // bm,),
        in_specs=[pl.BlockSpec((bm, K), lambda i: (i, 0)),
                  pl.BlockSpec((K, N), lambda i: (0, 0)),
                  pl.BlockSpec((1, N), lambda i: (0, 0))],
        out_specs=pl.BlockSpec((bm, N), lambda i: (i, 0)),
        out_shape=jax.ShapeDtypeStruct((M, N), jnp.float32),
    )(x, w, b.reshape(1, N))


def _mm5_body(x_ref, w_ref, b_ref, o0, o1, o2):
    y = jnp.dot(x_ref[...], w_ref[...], preferred_element_type=jnp.float32)
    y = y + b_ref[...]
    o0[...] = y[:, 0:2 * HID]
    o1[...] = y[:, 2 * HID:4 * HID]
    o2[...] = y[:, 4 * HID:5 * HID]


def _matmul5(x, w, b, bm=1000):
    M, K = x.shape
    kv = jax.ShapeDtypeStruct((M, 2 * HID), jnp.float32)
    return pl.pallas_call(
        _mm5_body,
        grid=(M // bm,),
        in_specs=[pl.BlockSpec((bm, K), lambda i: (i, 0)),
                  pl.BlockSpec((K, 5 * HID), lambda i: (0, 0)),
                  pl.BlockSpec((1, 5 * HID), lambda i: (0, 0))],
        out_specs=[pl.BlockSpec((bm, 2 * HID), lambda i: (i, 0)),
                   pl.BlockSpec((bm, 2 * HID), lambda i: (i, 0)),
                   pl.BlockSpec((bm, HID), lambda i: (i, 0))],
        out_shape=[kv, kv, jax.ShapeDtypeStruct((M, HID), jnp.float32)],
    )(x, w, b.reshape(1, 5 * HID))


def _msg_body(gkv_ref, gq_ref, d_ref, s_ref, r_ref, o_ref):
    gk = gkv_ref[:, :HID]
    gq = gq_ref[...]
    gv = gkv_ref[:, HID:]
    s = jnp.dot(gk * gq, s_ref[...], preferred_element_type=jnp.float32)
    e16 = jnp.exp(s)
    rep = jnp.dot(e16, r_ref[...], preferred_element_type=jnp.float32)
    df = lax.bitcast_convert_type(d_ref[...], jnp.float32)
    zpad = jnp.zeros((gk.shape[0], WMSG - HID - 17), jnp.float32)
    o_ref[...] = jnp.concatenate([gv * rep, e16, df, zpad], axis=1)


def _msg_kernel(gkv, gq, dst, smat, rmat, bm=512):
    E = gkv.shape[0]
    return pl.pallas_call(
        _msg_body,
        grid=(E // bm,),
        in_specs=[pl.BlockSpec((bm, 2 * HID), lambda i: (i, 0)),
                  pl.BlockSpec((bm, HID), lambda i: (i, 0)),
                  pl.BlockSpec((bm, 1), lambda i: (i, 0)),
                  pl.BlockSpec((HID, 16), lambda i: (0, 0)),
                  pl.BlockSpec((16, HID), lambda i: (0, 0))],
        out_specs=pl.BlockSpec((bm, WMSG), lambda i: (i, 0)),
        out_shape=jax.ShapeDtypeStruct((E, WMSG), jnp.float32),
    )(gkv, gq, dst.reshape(E, 1), smat, rmat)


def _outproj_body(agg_ref, h_ref, w_ref, b_ref, bv_ref, r4_ref, o_ref):
    a = agg_ref[...]
    den = a[:, HID:HID + 4]
    wgt = 1.0 / (den + 1e-16)
    wrep = jnp.dot(wgt, r4_ref[...], preferred_element_type=jnp.float32)
    x = a[:, :HID] * wrep
    g = jax.nn.gelu(x)
    o = jnp.dot(g, w_ref[...], preferred_element_type=jnp.float32) + b_ref[...]
    o_ref[...] = jnp.maximum(o + h_ref[...] * bv_ref[...], 0.0)


def _outproj(agg2, h_prev, w, b, bvec, r4, bm=1000):
    N = h_prev.shape[0]
    return pl.pallas_call(
        _outproj_body,
        grid=(N // bm,),
        in_specs=[pl.BlockSpec((bm, WAGG), lambda i: (i, 0)),
                  pl.BlockSpec((bm, HID), lambda i: (i, 0)),
                  pl.BlockSpec((HID, HID), lambda i: (0, 0)),
                  pl.BlockSpec((1, HID), lambda i: (0, 0)),
                  pl.BlockSpec((1, HID), lambda i: (0, 0)),
                  pl.BlockSpec((4, HID), lambda i: (0, 0))],
        out_specs=pl.BlockSpec((bm, HID), lambda i: (i, 0)),
        out_shape=jax.ShapeDtypeStruct((N, HID), jnp.float32),
    )(agg2, h_prev, w, b.reshape(1, HID), bvec.reshape(1, HID), r4)


def _head_body(ga_ref, gb_ref, ea_ref, wc_ref, b1_ref, w2_ref, b2_ref,
               w3_ref, b3_ref, o_ref, *, has_ea):
    z = ga_ref[...] + gb_ref[...] + b1_ref[...]
    if has_ea:
        z = z + jnp.dot(ea_ref[...], wc_ref[...],
                        preferred_element_type=jnp.float32)
    h = jnp.maximum(z, 0.0)
    h2 = jnp.dot(h, w2_ref[...], preferred_element_type=jnp.float32)
    h2 = jnp.maximum(h2 + b2_ref[...], 0.0)
    y = jnp.dot(h2, w3_ref[...], preferred_element_type=jnp.float32)
    o_ref[...] = y + b3_ref[...]


def _head(ga, gb, ea, wc, b1, w2, b2, w3, b3, bm=512):
    E = ga.shape[0]
    has_ea = ea is not None
    if not has_ea:
        ea = jnp.zeros((E, 8), jnp.float32)
        wc = jnp.zeros((8, HID), jnp.float32)
    return pl.pallas_call(
        functools.partial(_head_body, has_ea=has_ea),
        grid=(E // bm,),
        in_specs=[pl.BlockSpec((bm, HID), lambda i: (i, 0)),
                  pl.BlockSpec((bm, HID), lambda i: (i, 0)),
                  pl.BlockSpec((bm, 8), lambda i: (i, 0)),
                  pl.BlockSpec((8, HID), lambda i: (0, 0)),
                  pl.BlockSpec((1, HID), lambda i: (0, 0)),
                  pl.BlockSpec((HID, 128), lambda i: (0, 0)),
                  pl.BlockSpec((1, 128), lambda i: (0, 0)),
                  pl.BlockSpec((128, 1), lambda i: (0, 0)),
                  pl.BlockSpec((1, 1), lambda i: (0, 0))],
        out_specs=pl.BlockSpec((bm, 1), lambda i: (i, 0)),
        out_shape=jax.ShapeDtypeStruct((E, 1), jnp.float32),
    )(ga, gb, ea, wc, b1.reshape(1, HID), w2, b2.reshape(1, 128),
      w3, b3.reshape(1, 1))


def _sc_gather2(table1, idx1, table2, idx2):
    E = idx1.shape[0]
    W1 = table1.shape[1]
    W2 = table2.shape[1]
    cg = 64
    rows = E // NWORK
    nch = rows // cg
    assert idx2.shape[0] == E
    assert rows % cg == 0 and nch % 2 == 0 and W1 % 128 == 0 and W2 % 128 == 0
    mesh = plsc.VectorSubcoreMesh(core_axis_name="c", subcore_axis_name="s")

    @functools.partial(
        pl.kernel, mesh=mesh,
        out_type=[jax.ShapeDtypeStruct((E, W1), jnp.float32),
                  jax.ShapeDtypeStruct((E, W2), jnp.float32)],
        scratch_types=[
            pltpu.VMEM((rows,), jnp.int32),
            pltpu.VMEM((rows,), jnp.int32),
            pltpu.VMEM((cg, W1), jnp.float32),
            pltpu.VMEM((cg, W1), jnp.float32),
            pltpu.VMEM((cg, W2), jnp.float32),
            pltpu.VMEM((cg, W2), jnp.float32),
        ] + [pltpu.SemaphoreType.DMA] * 8,
    )
    def k(t1_hbm, i1_hbm, t2_hbm, i2_hbm, o1_hbm, o2_hbm,
          i1_v, i2_v, a1, b1, a2, b2, *sems):
        ga1, gb1, wa1, wb1, ga2, gb2, wa2, wb2 = sems
        wid = lax.axis_index("s") * 2 + lax.axis_index("c")
        base = wid * rows
        pltpu.sync_copy(i1_hbm.at[pl.ds(base, rows)], i1_v)
        pltpu.sync_copy(i2_hbm.at[pl.ds(base, rows)], i2_v)

        def start_g(tbl, iv, j, buf, sem):
            pltpu.async_copy(tbl.at[iv.at[pl.ds(j * cg, cg)]], buf, sem)

        def wait_g(tbl, iv, buf, sem):
            pltpu.make_async_copy(
                tbl.at[iv.at[pl.ds(0, cg)]], buf, sem).wait()

        def start_w(out, j, buf, sem):
            pltpu.async_copy(buf, out.at[pl.ds(base + j * cg, cg)], sem)

        def wait_w(out, buf, sem):
            pltpu.make_async_copy(buf, out.at[pl.ds(base, cg)], sem).wait()

        start_g(t1_hbm, i1_v, 0, a1, ga1)
        start_g(t2_hbm, i2_v, 0, a2, ga2)
        start_g(t1_hbm, i1_v, 1, b1, gb1)
        start_g(t2_hbm, i2_v, 1, b2, gb2)

        @pl.loop(0, nch // 2)
        def _(jj):
            j = jj * 2
            wait_g(t1_hbm, i1_v, a1, ga1)
            start_w(o1_hbm, j, a1, wa1)
            wait_g(t2_hbm, i2_v, a2, ga2)
            start_w(o2_hbm, j, a2, wa2)
            wait_g(t1_hbm, i1_v, b1, gb1)
            start_w(o1_hbm, j + 1, b1, wb1)
            wait_g(t2_hbm, i2_v, b2, gb2)
            start_w(o2_hbm, j + 1, b2, wb2)
            wait_w(o1_hbm, a1, wa1)
            start_g(t1_hbm, i1_v, lax.rem(j + 2, nch), a1, ga1)
            wait_w(o2_hbm, a2, wa2)
            start_g(t2_hbm, i2_v, lax.rem(j + 2, nch), a2, ga2)
            wait_w(o1_hbm, b1, wb1)
            start_g(t1_hbm, i1_v, lax.rem(j + 3, nch), b1, gb1)
            wait_w(o2_hbm, b2, wb2)
            start_g(t2_hbm, i2_v, lax.rem(j + 3, nch), b2, gb2)

        wait_g(t1_hbm, i1_v, a1, ga1)
        wait_g(t2_hbm, i2_v, a2, ga2)
        wait_g(t1_hbm, i1_v, b1, gb1)
        wait_g(t2_hbm, i2_v, b2, gb2)

    return k(table1, idx1, table2, idx2)


def _sc_segsum(msg, dst, nseg_pad, wrows, cap=8320):
    E = msg.shape[0]
    SC_CH = 2048
    nsc = E // SC_CH
    npass = nseg_pad // (NWORK * wrows)
    assert E % SC_CH == 0 and nseg_pad % (NWORK * wrows) == 0
    assert cap % 16 == 0
    mesh = plsc.VectorSubcoreMesh(core_axis_name="c", subcore_axis_name="s")
    cp = pltpu.CompilerParams()
    if "needs_layout_passes" in pltpu.CompilerParams.__dataclass_fields__:
        cp = dataclasses.replace(cp, needs_layout_passes=False)

    @functools.partial(
        pl.kernel, mesh=mesh, compiler_params=cp,
        out_type=jax.ShapeDtypeStruct((nseg_pad, WAGG), jnp.float32),
        scratch_types=[
            pltpu.VMEM((SC_CH,), jnp.int32),
            pltpu.VMEM((cap,), jnp.int32),
            pltpu.VMEM((CGATH, WMSG), jnp.float32),
            pltpu.VMEM((wrows, WAGG), jnp.float32),
            pltpu.SemaphoreType.DMA,
        ],
    )
    def k(msg_hbm, dst_hbm, out_hbm, sbuf, ids, gbuf, acc, gsem):
        w = lax.axis_index("s") * 2 + lax.axis_index("c")
        lane = lax.iota(jnp.int32, 16)
        zf16 = jnp.zeros((16,), jnp.float32)
        zi16 = jnp.zeros((16,), jnp.int32)

        @pl.loop(0, npass)
        def _(p):
            d0 = (p * NWORK + w) * wrows

            @pl.loop(0, wrows)
            def _(r):
                for t in range(WAGG // 16):
                    acc[r, pl.ds(t * 16, 16)] = zf16

            @pl.loop(0, cap // 16)
            def _(r):
                ids[pl.ds(r * 16, 16)] = zi16

            def scan_chunk(c, off):
                pltpu.sync_copy(dst_hbm.at[pl.ds(c * SC_CH, SC_CH)], sbuf)

                def scan_vec(t, off):
                    v = sbuf[pl.ds(t * 16, 16)]
                    m = (v >= d0) & (v < d0 + wrows)
                    eid = (c * SC_CH + t * 16) + lane
                    plsc.store_compressed(ids.at[pl.ds(off, 16)], eid, mask=m)
                    pc = plsc.all_reduce_population_count(m)
                    return off + jnp.max(pc)

                return lax.fori_loop(0, SC_CH // 16, scan_vec, off)

            off = lax.fori_loop(0, nsc, scan_chunk, jnp.int32(0))

            def gchunk(gc, _):
                pltpu.async_copy(
                    msg_hbm.at[ids.at[pl.ds(gc * CGATH, CGATH)]], gbuf,
                    gsem).wait()
                cnt = jnp.minimum(off - gc * CGATH, CGATH)

                def acc_edge(i, _):
                    q = gc * CGATH + i
                    dv = lax.bitcast_convert_type(
                        gbuf[i, pl.ds(HID + 16, 16)], jnp.int32)
                    d = jnp.max(jnp.where(lane == 0, dv, 0)) - d0
                    for t in range(WAGG // 16):
                        plsc.addupdate(acc.at[d, pl.ds(t * 16, 16)],
                                       gbuf[i, pl.ds(t * 16, 16)])
                    return 0

                lax.fori_loop(0, cnt, acc_edge, 0)
                return 0

            lax.fori_loop(0, (off + CGATH - 1) // CGATH, gchunk, 0)
            pltpu.sync_copy(acc, out_hbm.at[pl.ds(d0, wrows)])

    return k(msg, dst)


def _block_diag4(a):
    out = jnp.zeros((HID, HID), jnp.float32)
    for h in range(HEADS):
        out = out.at[h * DH:(h + 1) * DH, h * DH:(h + 1) * DH].set(a[h])
    return out


def _pad_idx(idx, epad, fill):
    return jnp.pad(idx, (0, epad - idx.shape[0]), constant_values=fill)


def kernel(x_sentence, x_criterion, x_post, edge_attr_supports, supports_src,
           supports_dst, next_src, next_dst, matches_src, matches_dst, params):
    p = params
    NS = x_sentence.shape[0]
    NC = x_criterion.shape[0]
    ES = supports_src.shape[0]
    EN = next_src.shape[0]
    EM = matches_src.shape[0]
    ES_P = 204800
    EN_P = 57344

    f32 = jnp.float32
    smat = (jnp.arange(HID)[:, None] // DH ==
            jnp.arange(16)[None, :]).astype(f32)
    rmat = (jnp.arange(16)[:, None] ==
            jnp.arange(HID)[None, :] // DH).astype(f32)
    r4 = rmat[:4]

    wq = {}
    wkv = {}
    wout = {}
    for l in range(2):
        for (edge, dt) in (('supports', 'criterion'), ('next', 'sentence')):
            scale = jnp.repeat(p[f'pr_{edge}_{l}'] / 8.0, DH)
            wq[(dt, l)] = (p[f'Wq_{dt}_{l}'] * scale[None, :],
                           p[f'bq_{dt}_{l}'] * scale)
        for edge in ('supports', 'next'):
            akb = _block_diag4(p[f'Ak_{edge}_{l}'])
            avb = _block_diag4(p[f'Av_{edge}_{l}'])
            wkv[(edge, 'k', l)] = (p[f'Wk_sentence_{l}'] @ akb,
                                   p[f'bk_sentence_{l}'] @ akb)
            wkv[(edge, 'v', l)] = (p[f'Wv_sentence_{l}'] @ avb,
                                   p[f'bv_sentence_{l}'] @ avb)
        for t in ('sentence', 'criterion'):
            alpha = jax.nn.sigmoid(p[f'skip_{t}_{l}'])
            wout[(t, l)] = (alpha * p[f'Wout_{t}_{l}'],
                            alpha * p[f'bout_{t}_{l}'],
                            (1.0 - alpha) * jnp.ones((HID,), f32))

    wcat = {}
    for l in range(2):
        ws = [wkv[('supports', 'k', l)], wkv[('supports', 'v', l)],
              wkv[('next', 'k', l)], wkv[('next', 'v', l)],
              wq[('sentence', l)]]
        wcat[l] = (jnp.concatenate([w for w, _ in ws], axis=1),
                   jnp.concatenate([b for _, b in ws], axis=0))

    s_src_g = _pad_idx(supports_src, ES_P, 0)
    s_dst_g = _pad_idx(supports_dst, ES_P, 0)
    s_dst_s = _pad_idx(supports_dst, ES_P, 2048)
    n_src_g = _pad_idx(next_src, EN_P, 0)
    n_dst_g = _pad_idx(next_dst, EN_P, 0)
    n_dst_s = _pad_idx(next_dst, EN_P, 51200)
    m_src_g = _pad_idx(matches_src, EN_P, 0)
    m_dst_g = _pad_idx(matches_dst, EN_P, 0)
    ea_pad = jnp.pad(edge_attr_supports, ((0, ES_P - ES), (0, 3)))

    h_s = _matmul(x_sentence, p['Wlin_sentence'], p['blin_sentence'], 'relu')
    h_c = _matmul(x_criterion, p['Wlin_criterion'], p['blin_criterion'], 'relu')
    h_p = _matmul(x_post, p['Wlin_post'], p['blin_post'], 'relu')

    for l in range(2):
        kv_sup, kv_nxt, q_s = _matmul5(h_s, *wcat[l])
        q_c = _matmul(h_c, *wq[('criterion', l)])

        gkv, gq = _sc_gather2(kv_sup, s_src_g, q_c, s_dst_g)
        msg_s = _msg_kernel(gkv, gq, s_dst_s, smat, rmat)
        agg_c = _sc_segsum(msg_s, s_dst_s, 2048, 64)

        gkv2, gq2 = _sc_gather2(kv_nxt, n_src_g, q_s, n_dst_g)
        msg_n = _msg_kernel(gkv2, gq2, n_dst_s, smat, rmat)
        agg_s = _sc_segsum(msg_n, n_dst_s, 51200, 160)

        h_c = _outproj(agg_c, h_c, *wout[('criterion', l)], r4)
        h_s = _outproj(agg_s, h_s, *wout[('sentence', l)], r4)

    we1a = p['We1'][:HID]
    we1b = p['We1'][HID:2 * HID]
    we1c = jnp.pad(p['We1'][2 * HID:], ((0, 3), (0, 0)))
    zb = jnp.zeros((HID,), f32)
    ea_a = _matmul(h_s, we1a, zb)
    ea_b = _matmul(h_c, we1b, zb)
    ga, gb = _sc_gather2(ea_a, s_src_g, ea_b, s_dst_g)
    edge_out = _head(ga, gb, ea_pad, we1c, p['be1'], p['We2'], p['be2'],
                     p['We3'], p['be3'])
    edge_logits = edge_out[:ES, 0]

    na_a = _matmul(h_p, p['Wn1'][:HID], zb)
    na_b = _matmul(h_c, p['Wn1'][HID:], zb)
    gp, gc = _sc_gather2(na_a, m_src_g, na_b, m_dst_g)
    node_out = _head(gp, gc, None, None, p['bn1'], p['Wn2'], p['bn2'],
                     p['Wn3'], p['bn3'])
    node_logits = node_out[:EM, 0]

    return (edge_logits, node_logits)


# speedup vs baseline: 1.9917x; 1.2121x over previous
import dataclasses
import functools
import jax
import jax.numpy as jnp
from jax import lax
from jax.experimental import pallas as pl
from jax.experimental.pallas import tpu as pltpu
from jax.experimental.pallas import tpu_sc as plsc

HID = 256
HEADS = 4
DH = 64
WMSG = 384
WAGG = 272
NWORK = 32
CGATH = 128


def _mm_body(x_ref, w_ref, b_ref, o_ref, *, act):
    y = jnp.dot(x_ref[...], w_ref[...], preferred_element_type=jnp.float32)
    y = y + b_ref[...]
    if act == 'relu':
        y = jnp.maximum(y, 0.0)
    o_ref[...] = y


def _matmul(x, w, b, act=None, bm=1000):
    M, K = x.shape
    N = w.shape[1]
    return pl.pallas_call(
        functools.partial(_mm_body, act=act),
        grid=(M // bm,),
        in_specs=[pl.BlockSpec((bm, K), lambda i: (i, 0)),
                  pl.BlockSpec((K, N), lambda i: (0, 0)),
                  pl.BlockSpec((1, N), lambda i: (0, 0))],
        out_specs=pl.BlockSpec((bm, N), lambda i: (i, 0)),
        out_shape=jax.ShapeDtypeStruct((M, N), jnp.float32),
    )(x, w, b.reshape(1, N))


def _mm5_body(x_ref, w_ref, b_ref, o0, o1, o2):
    y = jnp.dot(x_ref[...], w_ref[...], preferred_element_type=jnp.float32)
    y = y + b_ref[...]
    o0[...] = y[:, 0:2 * HID]
    o1[...] = y[:, 2 * HID:4 * HID]
    o2[...] = y[:, 4 * HID:5 * HID]


def _matmul5(x, w, b, bm=1000):
    M, K = x.shape
    kv = jax.ShapeDtypeStruct((M, 2 * HID), jnp.float32)
    return pl.pallas_call(
        _mm5_body,
        grid=(M // bm,),
        in_specs=[pl.BlockSpec((bm, K), lambda i: (i, 0)),
                  pl.BlockSpec((K, 5 * HID), lambda i: (0, 0)),
                  pl.BlockSpec((1, 5 * HID), lambda i: (0, 0))],
        out_specs=[pl.BlockSpec((bm, 2 * HID), lambda i: (i, 0)),
                   pl.BlockSpec((bm, 2 * HID), lambda i: (i, 0)),
                   pl.BlockSpec((bm, HID), lambda i: (i, 0))],
        out_shape=[kv, kv, jax.ShapeDtypeStruct((M, HID), jnp.float32)],
    )(x, w, b.reshape(1, 5 * HID))


def _msg_body(gkv_ref, gq_ref, d_ref, s_ref, r_ref, o_ref):
    gk = gkv_ref[:, :HID]
    gq = gq_ref[...]
    gv = gkv_ref[:, HID:]
    s = jnp.dot(gk * gq, s_ref[...], preferred_element_type=jnp.float32)
    e16 = jnp.exp(s)
    rep = jnp.dot(e16, r_ref[...], preferred_element_type=jnp.float32)
    df = lax.bitcast_convert_type(d_ref[...], jnp.float32)
    zpad = jnp.zeros((gk.shape[0], WMSG - HID - 17), jnp.float32)
    o_ref[...] = jnp.concatenate([gv * rep, e16, df, zpad], axis=1)


def _msg_kernel(gkv, gq, dst, smat, rmat, bm=512):
    E = gkv.shape[0]
    return pl.pallas_call(
        _msg_body,
        grid=(E // bm,),
        in_specs=[pl.BlockSpec((bm, 2 * HID), lambda i: (i, 0)),
                  pl.BlockSpec((bm, HID), lambda i: (i, 0)),
                  pl.BlockSpec((bm, 1), lambda i: (i, 0)),
                  pl.BlockSpec((HID, 16), lambda i: (0, 0)),
                  pl.BlockSpec((16, HID), lambda i: (0, 0))],
        out_specs=pl.BlockSpec((bm, WMSG), lambda i: (i, 0)),
        out_shape=jax.ShapeDtypeStruct((E, WMSG), jnp.float32),
    )(gkv, gq, dst.reshape(E, 1), smat, rmat)


def _outproj_body(agg_ref, h_ref, w_ref, b_ref, bv_ref, r4_ref, o_ref):
    a = agg_ref[...]
    den = a[:, HID:HID + 4]
    wgt = 1.0 / (den + 1e-16)
    wrep = jnp.dot(wgt, r4_ref[...], preferred_element_type=jnp.float32)
    x = a[:, :HID] * wrep
    g = jax.nn.gelu(x)
    o = jnp.dot(g, w_ref[...], preferred_element_type=jnp.float32) + b_ref[...]
    o_ref[...] = jnp.maximum(o + h_ref[...] * bv_ref[...], 0.0)


def _outproj(agg2, h_prev, w, b, bvec, r4, bm=1000):
    N = h_prev.shape[0]
    return pl.pallas_call(
        _outproj_body,
        grid=(N // bm,),
        in_specs=[pl.BlockSpec((bm, WAGG), lambda i: (i, 0)),
                  pl.BlockSpec((bm, HID), lambda i: (i, 0)),
                  pl.BlockSpec((HID, HID), lambda i: (0, 0)),
                  pl.BlockSpec((1, HID), lambda i: (0, 0)),
                  pl.BlockSpec((1, HID), lambda i: (0, 0)),
                  pl.BlockSpec((4, HID), lambda i: (0, 0))],
        out_specs=pl.BlockSpec((bm, HID), lambda i: (i, 0)),
        out_shape=jax.ShapeDtypeStruct((N, HID), jnp.float32),
    )(agg2, h_prev, w, b.reshape(1, HID), bvec.reshape(1, HID), r4)


def _head_body(ga_ref, gb_ref, ea_ref, wc_ref, b1_ref, w2_ref, b2_ref,
               w3_ref, b3_ref, o_ref, *, has_ea):
    z = ga_ref[...] + gb_ref[...] + b1_ref[...]
    if has_ea:
        z = z + jnp.dot(ea_ref[...], wc_ref[...],
                        preferred_element_type=jnp.float32)
    h = jnp.maximum(z, 0.0)
    h2 = jnp.dot(h, w2_ref[...], preferred_element_type=jnp.float32)
    h2 = jnp.maximum(h2 + b2_ref[...], 0.0)
    y = jnp.dot(h2, w3_ref[...], preferred_element_type=jnp.float32)
    o_ref[...] = y + b3_ref[...]


def _head(ga, gb, ea, wc, b1, w2, b2, w3, b3, bm=512):
    E = ga.shape[0]
    has_ea = ea is not None
    if not has_ea:
        ea = jnp.zeros((E, 8), jnp.float32)
        wc = jnp.zeros((8, HID), jnp.float32)
    return pl.pallas_call(
        functools.partial(_head_body, has_ea=has_ea),
        grid=(E // bm,),
        in_specs=[pl.BlockSpec((bm, HID), lambda i: (i, 0)),
                  pl.BlockSpec((bm, HID), lambda i: (i, 0)),
                  pl.BlockSpec((bm, 8), lambda i: (i, 0)),
                  pl.BlockSpec((8, HID), lambda i: (0, 0)),
                  pl.BlockSpec((1, HID), lambda i: (0, 0)),
                  pl.BlockSpec((HID, 128), lambda i: (0, 0)),
                  pl.BlockSpec((1, 128), lambda i: (0, 0)),
                  pl.BlockSpec((128, 1), lambda i: (0, 0)),
                  pl.BlockSpec((1, 1), lambda i: (0, 0))],
        out_specs=pl.BlockSpec((bm, 1), lambda i: (i, 0)),
        out_shape=jax.ShapeDtypeStruct((E, 1), jnp.float32),
    )(ga, gb, ea, wc, b1.reshape(1, HID), w2, b2.reshape(1, 128),
      w3, b3.reshape(1, 1))


def _sc_gather2(table1, idx1, table2, idx2):
    E = idx1.shape[0]
    W1 = table1.shape[1]
    W2 = table2.shape[1]
    cg = 64
    rows = E // NWORK
    nch = rows // cg
    assert idx2.shape[0] == E
    assert rows % cg == 0 and nch % 2 == 0 and W1 % 128 == 0 and W2 % 128 == 0
    mesh = plsc.VectorSubcoreMesh(core_axis_name="c", subcore_axis_name="s")

    @functools.partial(
        pl.kernel, mesh=mesh,
        out_type=[jax.ShapeDtypeStruct((E, W1), jnp.float32),
                  jax.ShapeDtypeStruct((E, W2), jnp.float32)],
        scratch_types=[
            pltpu.VMEM((rows,), jnp.int32),
            pltpu.VMEM((rows,), jnp.int32),
            pltpu.VMEM((cg, W1), jnp.float32),
            pltpu.VMEM((cg, W1), jnp.float32),
            pltpu.VMEM((cg, W2), jnp.float32),
            pltpu.VMEM((cg, W2), jnp.float32),
        ] + [pltpu.SemaphoreType.DMA] * 8,
    )
    def k(t1_hbm, i1_hbm, t2_hbm, i2_hbm, o1_hbm, o2_hbm,
          i1_v, i2_v, a1, b1, a2, b2, *sems):
        ga1, gb1, wa1, wb1, ga2, gb2, wa2, wb2 = sems
        wid = lax.axis_index("s") * 2 + lax.axis_index("c")
        base = wid * rows
        pltpu.sync_copy(i1_hbm.at[pl.ds(base, rows)], i1_v)
        pltpu.sync_copy(i2_hbm.at[pl.ds(base, rows)], i2_v)

        def start_g(tbl, iv, j, buf, sem):
            pltpu.async_copy(tbl.at[iv.at[pl.ds(j * cg, cg)]], buf, sem)

        def wait_g(tbl, iv, buf, sem):
            pltpu.make_async_copy(
                tbl.at[iv.at[pl.ds(0, cg)]], buf, sem).wait()

        def start_w(out, j, buf, sem):
            pltpu.async_copy(buf, out.at[pl.ds(base + j * cg, cg)], sem)

        def wait_w(out, buf, sem):
            pltpu.make_async_copy(buf, out.at[pl.ds(base, cg)], sem).wait()

        start_g(t1_hbm, i1_v, 0, a1, ga1)
        start_g(t2_hbm, i2_v, 0, a2, ga2)
        start_g(t1_hbm, i1_v, 1, b1, gb1)
        start_g(t2_hbm, i2_v, 1, b2, gb2)

        @pl.loop(0, nch // 2)
        def _(jj):
            j = jj * 2
            wait_g(t1_hbm, i1_v, a1, ga1)
            start_w(o1_hbm, j, a1, wa1)
            wait_g(t2_hbm, i2_v, a2, ga2)
            start_w(o2_hbm, j, a2, wa2)
            wait_g(t1_hbm, i1_v, b1, gb1)
            start_w(o1_hbm, j + 1, b1, wb1)
            wait_g(t2_hbm, i2_v, b2, gb2)
            start_w(o2_hbm, j + 1, b2, wb2)
            wait_w(o1_hbm, a1, wa1)
            start_g(t1_hbm, i1_v, lax.rem(j + 2, nch), a1, ga1)
            wait_w(o2_hbm, a2, wa2)
            start_g(t2_hbm, i2_v, lax.rem(j + 2, nch), a2, ga2)
            wait_w(o1_hbm, b1, wb1)
            start_g(t1_hbm, i1_v, lax.rem(j + 3, nch), b1, gb1)
            wait_w(o2_hbm, b2, wb2)
            start_g(t2_hbm, i2_v, lax.rem(j + 3, nch), b2, gb2)

        wait_g(t1_hbm, i1_v, a1, ga1)
        wait_g(t2_hbm, i2_v, a2, ga2)
        wait_g(t1_hbm, i1_v, b1, gb1)
        wait_g(t2_hbm, i2_v, b2, gb2)

    return k(table1, idx1, table2, idx2)


def _sc_segsum(msg, dst, nseg_pad, wrows, cap=8320):
    E = msg.shape[0]
    SC_CH = 2048
    nsc = E // SC_CH
    npass = nseg_pad // (NWORK * wrows)
    assert E % SC_CH == 0 and nseg_pad % (NWORK * wrows) == 0
    assert cap % 16 == 0
    mesh = plsc.VectorSubcoreMesh(core_axis_name="c", subcore_axis_name="s")
    cp = pltpu.CompilerParams()
    if "needs_layout_passes" in pltpu.CompilerParams.__dataclass_fields__:
        cp = dataclasses.replace(cp, needs_layout_passes=False)

    @functools.partial(
        pl.kernel, mesh=mesh, compiler_params=cp,
        out_type=jax.ShapeDtypeStruct((nseg_pad, WAGG), jnp.float32),
        scratch_types=[
            pltpu.VMEM((SC_CH,), jnp.int32),
            pltpu.VMEM((cap,), jnp.int32),
            pltpu.VMEM((64, WMSG), jnp.float32),
            pltpu.VMEM((64, WMSG), jnp.float32),
            pltpu.VMEM((wrows, WAGG), jnp.float32),
            pltpu.SemaphoreType.DMA,
            pltpu.SemaphoreType.DMA,
        ],
    )
    def k(msg_hbm, dst_hbm, out_hbm, sbuf, ids, gbuf_a, gbuf_b, acc,
          sem_a, sem_b):
        w = lax.axis_index("s") * 2 + lax.axis_index("c")
        lane = lax.iota(jnp.int32, 16)
        zf16 = jnp.zeros((16,), jnp.float32)
        zi16 = jnp.zeros((16,), jnp.int32)

        @pl.loop(0, npass)
        def _(p):
            d0 = (p * NWORK + w) * wrows

            @pl.loop(0, wrows)
            def _(r):
                for t in range(WAGG // 16):
                    acc[r, pl.ds(t * 16, 16)] = zf16

            @pl.loop(0, cap // 16)
            def _(r):
                ids[pl.ds(r * 16, 16)] = zi16

            def scan_chunk(c, off):
                pltpu.sync_copy(dst_hbm.at[pl.ds(c * SC_CH, SC_CH)], sbuf)

                def scan_vec(t, off):
                    v = sbuf[pl.ds(t * 16, 16)]
                    m = (v >= d0) & (v < d0 + wrows)
                    eid = (c * SC_CH + t * 16) + lane
                    plsc.store_compressed(ids.at[pl.ds(off, 16)], eid, mask=m)
                    pc = plsc.all_reduce_population_count(m)
                    return off + jnp.max(pc)

                return lax.fori_loop(0, SC_CH // 16, scan_vec, off)

            off = lax.fori_loop(0, nsc, scan_chunk, jnp.int32(0))

            CGS = 64
            ngc = (off + CGS - 1) // CGS

            def start_g(gc, buf, sem):
                pltpu.async_copy(
                    msg_hbm.at[ids.at[pl.ds(gc * CGS, CGS)]], buf, sem)

            def wait_g(buf, sem):
                pltpu.make_async_copy(
                    msg_hbm.at[ids.at[pl.ds(0, CGS)]], buf, sem).wait()

            def accum(gc, buf):
                cnt = jnp.minimum(off - gc * CGS, CGS)

                def acc_edge(i, _):
                    dv = lax.bitcast_convert_type(
                        buf[i, pl.ds(HID + 16, 16)], jnp.int32)
                    d = jnp.max(jnp.where(lane == 0, dv, 0)) - d0
                    for t in range(WAGG // 16):
                        plsc.addupdate(acc.at[d, pl.ds(t * 16, 16)],
                                       buf[i, pl.ds(t * 16, 16)])
                    return 0

                lax.fori_loop(0, cnt, acc_edge, 0)

            @pl.when(ngc > 0)
            def _():
                start_g(0, gbuf_a, sem_a)

            def pair(jj, _):
                j = 2 * jj

                @pl.when(j + 1 < ngc)
                def _():
                    start_g(j + 1, gbuf_b, sem_b)
                wait_g(gbuf_a, sem_a)
                accum(j, gbuf_a)

                @pl.when(j + 2 < ngc)
                def _():
                    start_g(j + 2, gbuf_a, sem_a)

                @pl.when(j + 1 < ngc)
                def _():
                    wait_g(gbuf_b, sem_b)
                    accum(j + 1, gbuf_b)
                return 0

            lax.fori_loop(0, (ngc + 1) // 2, pair, 0)
            pltpu.sync_copy(acc, out_hbm.at[pl.ds(d0, wrows)])

    return k(msg, dst)


def _block_diag4(a):
    out = jnp.zeros((HID, HID), jnp.float32)
    for h in range(HEADS):
        out = out.at[h * DH:(h + 1) * DH, h * DH:(h + 1) * DH].set(a[h])
    return out


def _pad_idx(idx, epad, fill):
    return jnp.pad(idx, (0, epad - idx.shape[0]), constant_values=fill)


def kernel(x_sentence, x_criterion, x_post, edge_attr_supports, supports_src,
           supports_dst, next_src, next_dst, matches_src, matches_dst, params):
    p = params
    NS = x_sentence.shape[0]
    NC = x_criterion.shape[0]
    ES = supports_src.shape[0]
    EN = next_src.shape[0]
    EM = matches_src.shape[0]
    ES_P = 204800
    EN_P = 57344

    f32 = jnp.float32
    smat = (jnp.arange(HID)[:, None] // DH ==
            jnp.arange(16)[None, :]).astype(f32)
    rmat = (jnp.arange(16)[:, None] ==
            jnp.arange(HID)[None, :] // DH).astype(f32)
    r4 = rmat[:4]

    wq = {}
    wkv = {}
    wout = {}
    for l in range(2):
        for (edge, dt) in (('supports', 'criterion'), ('next', 'sentence')):
            scale = jnp.repeat(p[f'pr_{edge}_{l}'] / 8.0, DH)
            wq[(dt, l)] = (p[f'Wq_{dt}_{l}'] * scale[None, :],
                           p[f'bq_{dt}_{l}'] * scale)
        for edge in ('supports', 'next'):
            akb = _block_diag4(p[f'Ak_{edge}_{l}'])
            avb = _block_diag4(p[f'Av_{edge}_{l}'])
            wkv[(edge, 'k', l)] = (p[f'Wk_sentence_{l}'] @ akb,
                                   p[f'bk_sentence_{l}'] @ akb)
            wkv[(edge, 'v', l)] = (p[f'Wv_sentence_{l}'] @ avb,
                                   p[f'bv_sentence_{l}'] @ avb)
        for t in ('sentence', 'criterion'):
            alpha = jax.nn.sigmoid(p[f'skip_{t}_{l}'])
            wout[(t, l)] = (alpha * p[f'Wout_{t}_{l}'],
                            alpha * p[f'bout_{t}_{l}'],
                            (1.0 - alpha) * jnp.ones((HID,), f32))

    wcat = {}
    for l in range(2):
        ws = [wkv[('supports', 'k', l)], wkv[('supports', 'v', l)],
              wkv[('next', 'k', l)], wkv[('next', 'v', l)],
              wq[('sentence', l)]]
        wcat[l] = (jnp.concatenate([w for w, _ in ws], axis=1),
                   jnp.concatenate([b for _, b in ws], axis=0))

    s_src_g = _pad_idx(supports_src, ES_P, 0)
    s_dst_g = _pad_idx(supports_dst, ES_P, 0)
    s_dst_s = _pad_idx(supports_dst, ES_P, 2048)
    n_src_g = _pad_idx(next_src, EN_P, 0)
    n_dst_g = _pad_idx(next_dst, EN_P, 0)
    n_dst_s = _pad_idx(next_dst, EN_P, 51200)
    m_src_g = _pad_idx(matches_src, EN_P, 0)
    m_dst_g = _pad_idx(matches_dst, EN_P, 0)
    ea_pad = jnp.pad(edge_attr_supports, ((0, ES_P - ES), (0, 3)))

    h_s = _matmul(x_sentence, p['Wlin_sentence'], p['blin_sentence'], 'relu')
    h_c = _matmul(x_criterion, p['Wlin_criterion'], p['blin_criterion'], 'relu')
    h_p = _matmul(x_post, p['Wlin_post'], p['blin_post'], 'relu')

    for l in range(2):
        kv_sup, kv_nxt, q_s = _matmul5(h_s, *wcat[l])
        q_c = _matmul(h_c, *wq[('criterion', l)])

        gkv, gq = _sc_gather2(kv_sup, s_src_g, q_c, s_dst_g)
        msg_s = _msg_kernel(gkv, gq, s_dst_s, smat, rmat)
        agg_c = _sc_segsum(msg_s, s_dst_s, 2048, 64)

        gkv2, gq2 = _sc_gather2(kv_nxt, n_src_g, q_s, n_dst_g)
        msg_n = _msg_kernel(gkv2, gq2, n_dst_s, smat, rmat)
        agg_s = _sc_segsum(msg_n, n_dst_s, 51200, 160)

        h_c = _outproj(agg_c, h_c, *wout[('criterion', l)], r4)
        h_s = _outproj(agg_s, h_s, *wout[('sentence', l)], r4)

    we1a = p['We1'][:HID]
    we1b = p['We1'][HID:2 * HID]
    we1c = jnp.pad(p['We1'][2 * HID:], ((0, 3), (0, 0)))
    zb = jnp.zeros((HID,), f32)
    ea_a = _matmul(h_s, we1a, zb)
    ea_b = _matmul(h_c, we1b, zb)
    ga, gb = _sc_gather2(ea_a, s_src_g, ea_b, s_dst_g)
    edge_out = _head(ga, gb, ea_pad, we1c, p['be1'], p['We2'], p['be2'],
                     p['We3'], p['be3'])
    edge_logits = edge_out[:ES, 0]

    na_a = _matmul(h_p, p['Wn1'][:HID], zb)
    na_b = _matmul(h_c, p['Wn1'][HID:], zb)
    gp, gc = _sc_gather2(na_a, m_src_g, na_b, m_dst_g)
    node_out = _head(gp, gc, None, None, p['bn1'], p['Wn2'], p['bn2'],
                     p['Wn3'], p['bn3'])
    node_logits = node_out[:EM, 0]

    return (edge_logits, node_logits)


# speedup vs baseline: 2.0836x; 1.0461x over previous
import dataclasses
import functools
import jax
import jax.numpy as jnp
from jax import lax
from jax.experimental import pallas as pl
from jax.experimental.pallas import tpu as pltpu
from jax.experimental.pallas import tpu_sc as plsc

HID = 256
HEADS = 4
DH = 64
WMSG = 384
WAGG = 272
NWORK = 32
CGATH = 128


def _mm_body(x_ref, w_ref, b_ref, o_ref, *, act):
    y = jnp.dot(x_ref[...], w_ref[...], preferred_element_type=jnp.float32)
    y = y + b_ref[...]
    if act == 'relu':
        y = jnp.maximum(y, 0.0)
    o_ref[...] = y


def _matmul(x, w, b, act=None, bm=1000):
    M, K = x.shape
    N = w.shape[1]
    return pl.pallas_call(
        functools.partial(_mm_body, act=act),
        grid=(M // bm,),
        in_specs=[pl.BlockSpec((bm, K), lambda i: (i, 0)),
                  pl.BlockSpec((K, N), lambda i: (0, 0)),
                  pl.BlockSpec((1, N), lambda i: (0, 0))],
        out_specs=pl.BlockSpec((bm, N), lambda i: (i, 0)),
        out_shape=jax.ShapeDtypeStruct((M, N), jnp.float32),
    )(x, w, b.reshape(1, N))


def _mm5_body(x_ref, w_ref, b_ref, o0, o1, o2):
    y = jnp.dot(x_ref[...], w_ref[...], preferred_element_type=jnp.float32)
    y = y + b_ref[...]
    o0[...] = y[:, 0:2 * HID]
    o1[...] = y[:, 2 * HID:4 * HID]
    o2[...] = y[:, 4 * HID:5 * HID]


def _matmul5(x, w, b, bm=1000):
    M, K = x.shape
    kv = jax.ShapeDtypeStruct((M, 2 * HID), jnp.float32)
    return pl.pallas_call(
        _mm5_body,
        grid=(M // bm,),
        in_specs=[pl.BlockSpec((bm, K), lambda i: (i, 0)),
                  pl.BlockSpec((K, 5 * HID), lambda i: (0, 0)),
                  pl.BlockSpec((1, 5 * HID), lambda i: (0, 0))],
        out_specs=[pl.BlockSpec((bm, 2 * HID), lambda i: (i, 0)),
                   pl.BlockSpec((bm, 2 * HID), lambda i: (i, 0)),
                   pl.BlockSpec((bm, HID), lambda i: (i, 0))],
        out_shape=[kv, kv, jax.ShapeDtypeStruct((M, HID), jnp.float32)],
    )(x, w, b.reshape(1, 5 * HID))


def _msg_body(gkv_ref, gq_ref, d_ref, s_ref, r_ref, o_ref):
    gk = gkv_ref[:, :HID]
    gq = gq_ref[...]
    gv = gkv_ref[:, HID:]
    s = jnp.dot(gk * gq, s_ref[...], preferred_element_type=jnp.float32)
    e16 = jnp.exp(s)
    rep = jnp.dot(e16, r_ref[...], preferred_element_type=jnp.float32)
    df = lax.bitcast_convert_type(d_ref[...], jnp.float32)
    zpad = jnp.zeros((gk.shape[0], WMSG - HID - 17), jnp.float32)
    o_ref[...] = jnp.concatenate([gv * rep, e16, df, zpad], axis=1)


def _msg_kernel(gkv, gq, dst, smat, rmat, bm=512):
    E = gkv.shape[0]
    return pl.pallas_call(
        _msg_body,
        grid=(E // bm,),
        in_specs=[pl.BlockSpec((bm, 2 * HID), lambda i: (i, 0)),
                  pl.BlockSpec((bm, HID), lambda i: (i, 0)),
                  pl.BlockSpec((bm, 1), lambda i: (i, 0)),
                  pl.BlockSpec((HID, 16), lambda i: (0, 0)),
                  pl.BlockSpec((16, HID), lambda i: (0, 0))],
        out_specs=pl.BlockSpec((bm, WMSG), lambda i: (i, 0)),
        out_shape=jax.ShapeDtypeStruct((E, WMSG), jnp.float32),
    )(gkv, gq, dst.reshape(E, 1), smat, rmat)


def _outproj_body(agg_ref, h_ref, w_ref, b_ref, bv_ref, r4_ref, o_ref):
    a = agg_ref[...]
    den = a[:, HID:HID + 4]
    wgt = 1.0 / (den + 1e-16)
    wrep = jnp.dot(wgt, r4_ref[...], preferred_element_type=jnp.float32)
    x = a[:, :HID] * wrep
    g = jax.nn.gelu(x)
    o = jnp.dot(g, w_ref[...], preferred_element_type=jnp.float32) + b_ref[...]
    o_ref[...] = jnp.maximum(o + h_ref[...] * bv_ref[...], 0.0)


def _outproj(agg2, h_prev, w, b, bvec, r4, bm=1000):
    N = h_prev.shape[0]
    return pl.pallas_call(
        _outproj_body,
        grid=(N // bm,),
        in_specs=[pl.BlockSpec((bm, WAGG), lambda i: (i, 0)),
                  pl.BlockSpec((bm, HID), lambda i: (i, 0)),
                  pl.BlockSpec((HID, HID), lambda i: (0, 0)),
                  pl.BlockSpec((1, HID), lambda i: (0, 0)),
                  pl.BlockSpec((1, HID), lambda i: (0, 0)),
                  pl.BlockSpec((4, HID), lambda i: (0, 0))],
        out_specs=pl.BlockSpec((bm, HID), lambda i: (i, 0)),
        out_shape=jax.ShapeDtypeStruct((N, HID), jnp.float32),
    )(agg2, h_prev, w, b.reshape(1, HID), bvec.reshape(1, HID), r4)


def _head_body(ga_ref, gb_ref, ea_ref, wc_ref, b1_ref, w2_ref, b2_ref,
               w3_ref, b3_ref, o_ref, *, has_ea):
    z = ga_ref[...] + gb_ref[...] + b1_ref[...]
    if has_ea:
        z = z + jnp.dot(ea_ref[...], wc_ref[...],
                        preferred_element_type=jnp.float32)
    h = jnp.maximum(z, 0.0)
    h2 = jnp.dot(h, w2_ref[...], preferred_element_type=jnp.float32)
    h2 = jnp.maximum(h2 + b2_ref[...], 0.0)
    y = jnp.dot(h2, w3_ref[...], preferred_element_type=jnp.float32)
    o_ref[...] = y + b3_ref[...]


def _head(ga, gb, ea, wc, b1, w2, b2, w3, b3, bm=512):
    E = ga.shape[0]
    has_ea = ea is not None
    if not has_ea:
        ea = jnp.zeros((E, 8), jnp.float32)
        wc = jnp.zeros((8, HID), jnp.float32)
    return pl.pallas_call(
        functools.partial(_head_body, has_ea=has_ea),
        grid=(E // bm,),
        in_specs=[pl.BlockSpec((bm, HID), lambda i: (i, 0)),
                  pl.BlockSpec((bm, HID), lambda i: (i, 0)),
                  pl.BlockSpec((bm, 8), lambda i: (i, 0)),
                  pl.BlockSpec((8, HID), lambda i: (0, 0)),
                  pl.BlockSpec((1, HID), lambda i: (0, 0)),
                  pl.BlockSpec((HID, 128), lambda i: (0, 0)),
                  pl.BlockSpec((1, 128), lambda i: (0, 0)),
                  pl.BlockSpec((128, 1), lambda i: (0, 0)),
                  pl.BlockSpec((1, 1), lambda i: (0, 0))],
        out_specs=pl.BlockSpec((bm, 1), lambda i: (i, 0)),
        out_shape=jax.ShapeDtypeStruct((E, 1), jnp.float32),
    )(ga, gb, ea, wc, b1.reshape(1, HID), w2, b2.reshape(1, 128),
      w3, b3.reshape(1, 1))


def _sc_gather2(table1, idx1, table2, idx2):
    E = idx1.shape[0]
    W1 = table1.shape[1]
    W2 = table2.shape[1]
    cg = 64
    rows = E // NWORK
    nch = rows // cg
    assert idx2.shape[0] == E
    assert rows % cg == 0 and nch % 2 == 0 and W1 % 128 == 0 and W2 % 128 == 0
    mesh = plsc.VectorSubcoreMesh(core_axis_name="c", subcore_axis_name="s")

    @functools.partial(
        pl.kernel, mesh=mesh,
        out_type=[jax.ShapeDtypeStruct((E, W1), jnp.float32),
                  jax.ShapeDtypeStruct((E, W2), jnp.float32)],
        scratch_types=[
            pltpu.VMEM((rows,), jnp.int32),
            pltpu.VMEM((rows,), jnp.int32),
            pltpu.VMEM((cg, W1), jnp.float32),
            pltpu.VMEM((cg, W1), jnp.float32),
            pltpu.VMEM((cg, W2), jnp.float32),
            pltpu.VMEM((cg, W2), jnp.float32),
        ] + [pltpu.SemaphoreType.DMA] * 8,
    )
    def k(t1_hbm, i1_hbm, t2_hbm, i2_hbm, o1_hbm, o2_hbm,
          i1_v, i2_v, a1, b1, a2, b2, *sems):
        ga1, gb1, wa1, wb1, ga2, gb2, wa2, wb2 = sems
        wid = lax.axis_index("s") * 2 + lax.axis_index("c")
        base = wid * rows
        pltpu.sync_copy(i1_hbm.at[pl.ds(base, rows)], i1_v)
        pltpu.sync_copy(i2_hbm.at[pl.ds(base, rows)], i2_v)

        def start_g(tbl, iv, j, buf, sem):
            pltpu.async_copy(tbl.at[iv.at[pl.ds(j * cg, cg)]], buf, sem)

        def wait_g(tbl, iv, buf, sem):
            pltpu.make_async_copy(
                tbl.at[iv.at[pl.ds(0, cg)]], buf, sem).wait()

        def start_w(out, j, buf, sem):
            pltpu.async_copy(buf, out.at[pl.ds(base + j * cg, cg)], sem)

        def wait_w(out, buf, sem):
            pltpu.make_async_copy(buf, out.at[pl.ds(base, cg)], sem).wait()

        start_g(t1_hbm, i1_v, 0, a1, ga1)
        start_g(t2_hbm, i2_v, 0, a2, ga2)
        start_g(t1_hbm, i1_v, 1, b1, gb1)
        start_g(t2_hbm, i2_v, 1, b2, gb2)

        @pl.loop(0, nch // 2)
        def _(jj):
            j = jj * 2
            wait_g(t1_hbm, i1_v, a1, ga1)
            start_w(o1_hbm, j, a1, wa1)
            wait_g(t2_hbm, i2_v, a2, ga2)
            start_w(o2_hbm, j, a2, wa2)
            wait_g(t1_hbm, i1_v, b1, gb1)
            start_w(o1_hbm, j + 1, b1, wb1)
            wait_g(t2_hbm, i2_v, b2, gb2)
            start_w(o2_hbm, j + 1, b2, wb2)
            wait_w(o1_hbm, a1, wa1)
            start_g(t1_hbm, i1_v, lax.rem(j + 2, nch), a1, ga1)
            wait_w(o2_hbm, a2, wa2)
            start_g(t2_hbm, i2_v, lax.rem(j + 2, nch), a2, ga2)
            wait_w(o1_hbm, b1, wb1)
            start_g(t1_hbm, i1_v, lax.rem(j + 3, nch), b1, gb1)
            wait_w(o2_hbm, b2, wb2)
            start_g(t2_hbm, i2_v, lax.rem(j + 3, nch), b2, gb2)

        wait_g(t1_hbm, i1_v, a1, ga1)
        wait_g(t2_hbm, i2_v, a2, ga2)
        wait_g(t1_hbm, i1_v, b1, gb1)
        wait_g(t2_hbm, i2_v, b2, gb2)

    return k(table1, idx1, table2, idx2)


def _sc_segsum(msg, dst, nseg_pad, wrows, cap=8320):
    E = msg.shape[0]
    SC_CH = 2048
    nsc = E // SC_CH
    npass = nseg_pad // (NWORK * wrows)
    assert E % SC_CH == 0 and nsc % 2 == 0
    assert nseg_pad % (NWORK * wrows) == 0
    assert cap % 16 == 0
    mesh = plsc.VectorSubcoreMesh(core_axis_name="c", subcore_axis_name="s")
    cp = pltpu.CompilerParams()
    if "needs_layout_passes" in pltpu.CompilerParams.__dataclass_fields__:
        cp = dataclasses.replace(cp, needs_layout_passes=False)

    @functools.partial(
        pl.kernel, mesh=mesh, compiler_params=cp,
        out_type=jax.ShapeDtypeStruct((nseg_pad, WAGG), jnp.float32),
        scratch_types=[
            pltpu.VMEM((SC_CH,), jnp.int32),
            pltpu.VMEM((SC_CH,), jnp.int32),
            pltpu.VMEM((cap,), jnp.int32),
            pltpu.VMEM((64, WMSG), jnp.float32),
            pltpu.VMEM((64, WMSG), jnp.float32),
            pltpu.VMEM((wrows, WAGG), jnp.float32),
            pltpu.SemaphoreType.DMA,
            pltpu.SemaphoreType.DMA,
        ],
    )
    def k(msg_hbm, dst_hbm, out_hbm, sbuf_a, sbuf_b, ids, gbuf_a, gbuf_b,
          acc, sem_a, sem_b):
        w = lax.axis_index("s") * 2 + lax.axis_index("c")
        lane = lax.iota(jnp.int32, 16)
        zf16 = jnp.zeros((16,), jnp.float32)
        zi16 = jnp.zeros((16,), jnp.int32)

        @pl.loop(0, npass)
        def _(p):
            d0 = (p * NWORK + w) * wrows

            @pl.loop(0, wrows)
            def _(r):
                for t in range(WAGG // 16):
                    acc[r, pl.ds(t * 16, 16)] = zf16

            @pl.loop(0, cap // 16)
            def _(r):
                ids[pl.ds(r * 16, 16)] = zi16

            def start_s(c, buf, sem):
                pltpu.async_copy(dst_hbm.at[pl.ds(c * SC_CH, SC_CH)],
                                 buf, sem)

            def wait_s(buf, sem):
                pltpu.make_async_copy(dst_hbm.at[pl.ds(0, SC_CH)],
                                      buf, sem).wait()

            def scan_buf(c, buf, off):
                def scan_vec(t, off):
                    v = buf[pl.ds(t * 16, 16)]
                    m = (v >= d0) & (v < d0 + wrows)
                    eid = (c * SC_CH + t * 16) + lane
                    plsc.store_compressed(ids.at[pl.ds(off, 16)], eid, mask=m)
                    pc = plsc.all_reduce_population_count(m)
                    return off + jnp.max(pc)

                return lax.fori_loop(0, SC_CH // 16, scan_vec, off)

            start_s(0, sbuf_a, sem_a)

            def scan_pair(jj, off):
                c = 2 * jj
                if True:
                    start_s(c + 1, sbuf_b, sem_b)
                wait_s(sbuf_a, sem_a)
                off = scan_buf(c, sbuf_a, off)
                start_s(lax.rem(c + 2, nsc), sbuf_a, sem_a)
                wait_s(sbuf_b, sem_b)
                off = scan_buf(c + 1, sbuf_b, off)
                return off

            off = lax.fori_loop(0, nsc // 2, scan_pair, jnp.int32(0))
            wait_s(sbuf_a, sem_a)

            CGS = 64
            ngc = (off + CGS - 1) // CGS

            def start_g(gc, buf, sem):
                pltpu.async_copy(
                    msg_hbm.at[ids.at[pl.ds(gc * CGS, CGS)]], buf, sem)

            def wait_g(buf, sem):
                pltpu.make_async_copy(
                    msg_hbm.at[ids.at[pl.ds(0, CGS)]], buf, sem).wait()

            def accum(gc, buf):
                cnt = jnp.minimum(off - gc * CGS, CGS)

                def acc_edge(i, _):
                    dv = lax.bitcast_convert_type(
                        buf[i, pl.ds(HID + 16, 16)], jnp.int32)
                    d = jnp.max(jnp.where(lane == 0, dv, 0)) - d0
                    for t in range(WAGG // 16):
                        plsc.addupdate(acc.at[d, pl.ds(t * 16, 16)],
                                       buf[i, pl.ds(t * 16, 16)])
                    return 0

                lax.fori_loop(0, cnt, acc_edge, 0)

            @pl.when(ngc > 0)
            def _():
                start_g(0, gbuf_a, sem_a)

            def pair(jj, _):
                j = 2 * jj

                @pl.when(j + 1 < ngc)
                def _():
                    start_g(j + 1, gbuf_b, sem_b)
                wait_g(gbuf_a, sem_a)
                accum(j, gbuf_a)

                @pl.when(j + 2 < ngc)
                def _():
                    start_g(j + 2, gbuf_a, sem_a)

                @pl.when(j + 1 < ngc)
                def _():
                    wait_g(gbuf_b, sem_b)
                    accum(j + 1, gbuf_b)
                return 0

            lax.fori_loop(0, (ngc + 1) // 2, pair, 0)
            pltpu.sync_copy(acc, out_hbm.at[pl.ds(d0, wrows)])

    return k(msg, dst)


def _block_diag4(a):
    out = jnp.zeros((HID, HID), jnp.float32)
    for h in range(HEADS):
        out = out.at[h * DH:(h + 1) * DH, h * DH:(h + 1) * DH].set(a[h])
    return out


def _pad_idx(idx, epad, fill):
    return jnp.pad(idx, (0, epad - idx.shape[0]), constant_values=fill)


def kernel(x_sentence, x_criterion, x_post, edge_attr_supports, supports_src,
           supports_dst, next_src, next_dst, matches_src, matches_dst, params):
    p = params
    NS = x_sentence.shape[0]
    NC = x_criterion.shape[0]
    ES = supports_src.shape[0]
    EN = next_src.shape[0]
    EM = matches_src.shape[0]
    ES_P = 204800
    EN_P = 57344

    f32 = jnp.float32
    smat = (jnp.arange(HID)[:, None] // DH ==
            jnp.arange(16)[None, :]).astype(f32)
    rmat = (jnp.arange(16)[:, None] ==
            jnp.arange(HID)[None, :] // DH).astype(f32)
    r4 = rmat[:4]

    wq = {}
    wkv = {}
    wout = {}
    for l in range(2):
        for (edge, dt) in (('supports', 'criterion'), ('next', 'sentence')):
            scale = jnp.repeat(p[f'pr_{edge}_{l}'] / 8.0, DH)
            wq[(dt, l)] = (p[f'Wq_{dt}_{l}'] * scale[None, :],
                           p[f'bq_{dt}_{l}'] * scale)
        for edge in ('supports', 'next'):
            akb = _block_diag4(p[f'Ak_{edge}_{l}'])
            avb = _block_diag4(p[f'Av_{edge}_{l}'])
            wkv[(edge, 'k', l)] = (p[f'Wk_sentence_{l}'] @ akb,
                                   p[f'bk_sentence_{l}'] @ akb)
            wkv[(edge, 'v', l)] = (p[f'Wv_sentence_{l}'] @ avb,
                                   p[f'bv_sentence_{l}'] @ avb)
        for t in ('sentence', 'criterion'):
            alpha = jax.nn.sigmoid(p[f'skip_{t}_{l}'])
            wout[(t, l)] = (alpha * p[f'Wout_{t}_{l}'],
                            alpha * p[f'bout_{t}_{l}'],
                            (1.0 - alpha) * jnp.ones((HID,), f32))

    wcat = {}
    for l in range(2):
        ws = [wkv[('supports', 'k', l)], wkv[('supports', 'v', l)],
              wkv[('next', 'k', l)], wkv[('next', 'v', l)],
              wq[('sentence', l)]]
        wcat[l] = (jnp.concatenate([w for w, _ in ws], axis=1),
                   jnp.concatenate([b for _, b in ws], axis=0))

    s_src_g = _pad_idx(supports_src, ES_P, 0)
    s_dst_g = _pad_idx(supports_dst, ES_P, 0)
    s_dst_s = _pad_idx(supports_dst, ES_P, 2048)
    n_src_g = _pad_idx(next_src, EN_P, 0)
    n_dst_g = _pad_idx(next_dst, EN_P, 0)
    n_dst_s = _pad_idx(next_dst, EN_P, 51200)
    m_src_g = _pad_idx(matches_src, EN_P, 0)
    m_dst_g = _pad_idx(matches_dst, EN_P, 0)
    ea_pad = jnp.pad(edge_attr_supports, ((0, ES_P - ES), (0, 3)))

    h_s = _matmul(x_sentence, p['Wlin_sentence'], p['blin_sentence'], 'relu')
    h_c = _matmul(x_criterion, p['Wlin_criterion'], p['blin_criterion'], 'relu')
    h_p = _matmul(x_post, p['Wlin_post'], p['blin_post'], 'relu')

    for l in range(2):
        kv_sup, kv_nxt, q_s = _matmul5(h_s, *wcat[l])
        q_c = _matmul(h_c, *wq[('criterion', l)])

        gkv, gq = _sc_gather2(kv_sup, s_src_g, q_c, s_dst_g)
        msg_s = _msg_kernel(gkv, gq, s_dst_s, smat, rmat)
        agg_c = _sc_segsum(msg_s, s_dst_s, 2048, 64)

        gkv2, gq2 = _sc_gather2(kv_nxt, n_src_g, q_s, n_dst_g)
        msg_n = _msg_kernel(gkv2, gq2, n_dst_s, smat, rmat)
        agg_s = _sc_segsum(msg_n, n_dst_s, 51200, 160)

        h_c = _outproj(agg_c, h_c, *wout[('criterion', l)], r4)
        h_s = _outproj(agg_s, h_s, *wout[('sentence', l)], r4)

    we1a = p['We1'][:HID]
    we1b = p['We1'][HID:2 * HID]
    we1c = jnp.pad(p['We1'][2 * HID:], ((0, 3), (0, 0)))
    zb = jnp.zeros((HID,), f32)
    ea_a = _matmul(h_s, we1a, zb)
    ea_b = _matmul(h_c, we1b, zb)
    ga, gb = _sc_gather2(ea_a, s_src_g, ea_b, s_dst_g)
    edge_out = _head(ga, gb, ea_pad, we1c, p['be1'], p['We2'], p['be2'],
                     p['We3'], p['be3'])
    edge_logits = edge_out[:ES, 0]

    na_a = _matmul(h_p, p['Wn1'][:HID], zb)
    na_b = _matmul(h_c, p['Wn1'][HID:], zb)
    gp, gc = _sc_gather2(na_a, m_src_g, na_b, m_dst_g)
    node_out = _head(gp, gc, None, None, p['bn1'], p['Wn2'], p['bn2'],
                     p['Wn3'], p['bn3'])
    node_logits = node_out[:EM, 0]

    return (edge_logits, node_logits)
